# baseline (speedup 1.0000x reference)
import functools

import jax
import jax.numpy as jnp
from jax import lax
from jax.experimental import pallas as pl
from jax.experimental.pallas import tpu as pltpu
from jax.experimental.pallas import tpu_sc as plsc

D_MODEL = 1024
EMBED_LEN = 8192
NC = 2
NS = 16
NW = NC * NS
CHUNK = 32
NBUF = 3


def _pe_gather(positions_hbm, table_hbm, out_hbm, idx_v, bufs, gsems, ssems):
    n_total = out_hbm.shape[0]
    b_per_w = n_total // NW
    n_groups = b_per_w // (NBUF * CHUNK)

    wid = lax.axis_index("s") * NC + lax.axis_index("c")
    base = wid * b_per_w

    pltpu.sync_copy(positions_hbm.at[pl.ds(base, b_per_w)], idx_v)

    def gather(i, b):
        off = pl.multiple_of(i * CHUNK, CHUNK)
        pltpu.async_copy(table_hbm.at[idx_v.at[pl.ds(off, CHUNK)]],
                         bufs[b], gsems[b])

    def wait_gather(b):
        pltpu.make_async_copy(table_hbm.at[idx_v.at[pl.ds(0, CHUNK)]],
                              bufs[b], gsems[b]).wait()

    def store(i, b):
        off = pl.multiple_of(i * CHUNK, CHUNK)
        pltpu.async_copy(bufs[b], out_hbm.at[pl.ds(base + off, CHUNK)],
                         ssems[b])

    def wait_store(b):
        pltpu.make_async_copy(bufs[b],
                              out_hbm.at[pl.ds(base, CHUNK)], ssems[b]).wait()

    def store_sync(i, b):
        off = pl.multiple_of(i * CHUNK, CHUNK)
        pltpu.sync_copy(bufs[b], out_hbm.at[pl.ds(base + off, CHUNK)])

    n_chunks = b_per_w // CHUNK
    n_grps = n_chunks // NBUF
    n_epi = n_chunks - n_grps * NBUF

    for b in range(NBUF):
        gather(b, b)

    def body(p, carry):
        i0 = NBUF * p
        for b in range(NBUF):
            wait_gather(b)
            store(i0 + b, b)
            wait_store(b)

            @pl.when(i0 + NBUF + b < n_chunks)
            def _():
                gather(i0 + NBUF + b, b)
        return carry

    lax.fori_loop(0, n_grps, body, 0)

    for b in range(n_epi):
        wait_gather(b)
        store(n_grps * NBUF + b, b)
        wait_store(b)


@jax.jit
def _pe_lookup(positions_flat, pe_table):
    n_total = positions_flat.shape[0]
    mesh = plsc.VectorSubcoreMesh(core_axis_name="c", subcore_axis_name="s")
    k = pl.kernel(
        _pe_gather,
        out_type=jax.ShapeDtypeStruct((n_total, D_MODEL), jnp.float32),
        mesh=mesh,
        scratch_types=[
            pltpu.VMEM((n_total // NW,), jnp.int32),
            [pltpu.VMEM((CHUNK, D_MODEL), jnp.float32)] * NBUF,
            [pltpu.SemaphoreType.DMA] * NBUF,
            [pltpu.SemaphoreType.DMA] * NBUF,
        ],
    )
    return k(positions_flat, pe_table)


def kernel(positions, pe_table):
    b, s = positions.shape
    out = _pe_lookup(positions.reshape(b * s), pe_table)
    return out.reshape(b, s, pe_table.shape[1])


# speedup vs baseline: 1.0021x; 1.0021x over previous
import functools

import jax
import jax.numpy as jnp
from jax import lax
from jax.experimental import pallas as pl
from jax.experimental.pallas import tpu as pltpu
from jax.experimental.pallas import tpu_sc as plsc

D_MODEL = 1024
EMBED_LEN = 8192
NC = 2
NS = 16
NW = NC * NS
CHUNK = 16
NBUF = 2


def _pe_gather(positions_hbm, table_hbm, out_hbm, idx_v, bufs, spm, gsems,
               ssems):
    n_total = out_hbm.shape[0]
    b_per_w = n_total // NW
    n_groups = b_per_w // (NBUF * CHUNK)

    sid = lax.axis_index("s")
    wid = sid * NC + lax.axis_index("c")
    base = wid * b_per_w

    pltpu.sync_copy(positions_hbm.at[pl.ds(base, b_per_w)], idx_v)

    def gather(i, b):
        off = pl.multiple_of(i * CHUNK, CHUNK)
        pltpu.async_copy(table_hbm.at[idx_v.at[pl.ds(off, CHUNK)]],
                         bufs[b], gsems[b])

    def wait_gather(b):
        pltpu.make_async_copy(table_hbm.at[idx_v.at[pl.ds(0, CHUNK)]],
                              bufs[b], gsems[b]).wait()

    def store(i, b):
        off = pl.multiple_of(i * CHUNK, CHUNK)
        pltpu.async_copy(bufs[b], out_hbm.at[pl.ds(base + off, CHUNK)],
                         ssems[b])

    def wait_store(b):
        pltpu.make_async_copy(bufs[b],
                              out_hbm.at[pl.ds(base, CHUNK)], ssems[b]).wait()

    def store_sync(i, b):
        off = pl.multiple_of(i * CHUNK, CHUNK)
        pltpu.sync_copy(bufs[b], out_hbm.at[pl.ds(base + off, CHUNK)])

    def store_spm(i, b):
        off = pl.multiple_of(i * CHUNK, CHUNK)
        pltpu.async_copy(spm.at[sid, b], out_hbm.at[pl.ds(base + off, CHUNK)],
                         ssems[b])

    def wait_store_spm(b):
        pltpu.make_async_copy(spm.at[sid, b],
                              out_hbm.at[pl.ds(base, CHUNK)], ssems[b]).wait()

    n_chunks = b_per_w // CHUNK
    n_pairs = n_chunks // NBUF

    gather(0, 0)
    gather(1, 1)

    def body(p, carry):
        i0 = NBUF * p
        for b in range(NBUF):
            i = i0 + b
            wait_gather(b)

            @pl.when(p > 0)
            def _():
                wait_store_spm(b)

            pltpu.sync_copy(bufs[b], spm.at[sid, b])

            @pl.when(i + NBUF < n_chunks)
            def _():
                gather(i + NBUF, b)

            store_spm(i, b)
        return carry

    lax.fori_loop(0, n_pairs, body, 0)
    for b in range(NBUF):
        wait_store_spm(b)


@jax.jit
def _pe_lookup(positions_flat, pe_table):
    n_total = positions_flat.shape[0]
    mesh = plsc.VectorSubcoreMesh(core_axis_name="c", subcore_axis_name="s")
    k = pl.kernel(
        _pe_gather,
        out_type=jax.ShapeDtypeStruct((n_total, D_MODEL), jnp.float32),
        mesh=mesh,
        scratch_types=[
            pltpu.VMEM((n_total // NW,), jnp.int32),
            [pltpu.VMEM((CHUNK, D_MODEL), jnp.float32)] * NBUF,
            pltpu.VMEM_SHARED((NS, NBUF, CHUNK, D_MODEL), jnp.float32),
            [pltpu.SemaphoreType.DMA] * NBUF,
            [pltpu.SemaphoreType.DMA] * NBUF,
        ],
    )
    return k(positions_flat, pe_table)


def kernel(positions, pe_table):
    b, s = positions.shape
    out = _pe_lookup(positions.reshape(b * s), pe_table)
    return out.reshape(b, s, pe_table.shape[1])
